# Initial kernel scaffold; baseline (speedup 1.0000x reference)
import jax
import jax.numpy as jnp
from jax.experimental import pallas as pl


def kernel(node_emb, W1, root1, b1, W2, root2, b2, rel_emb, edge_index, edge_type):
    raise NotImplementedError("write your pallas kernel here")


# speedup vs baseline: 2.8383x; 2.8383x over previous
import functools

import jax
import jax.numpy as jnp
from jax import lax
from jax.experimental import pallas as pl
from jax.experimental.pallas import tpu as pltpu
from jax.experimental.pallas import tpu_sc as plsc

N = 10000
R = 30
H = 200
NB = 5
BS = 40
E = 320000

HPAD = 256
HH = 128
NPAD = 10240
NR = N * R
NRPAD = 307200
NTILES = 32
CH = 80
EPT32 = E // 32
EPT16 = E // 16
NAGG = 10048
ROWS_PS = 1256
NSL = 13

_MESH = plsc.VectorSubcoreMesh(core_axis_name="c", subcore_axis_name="s")


def _f32(shape):
    return jax.ShapeDtypeStruct(shape, jnp.float32)


@functools.partial(
    pl.kernel,
    mesh=_MESH,
    out_type=_f32((2, NRPAD)),
    scratch_types=[
        pltpu.VMEM((CH,), jnp.int32),
        pltpu.VMEM((CH,), jnp.int32),
        pltpu.VMEM((CH,), jnp.int32),
        pltpu.VMEM((CH,), jnp.float32),
        pltpu.VMEM_SHARED((NRPAD,), jnp.float32),
    ],
)
def _sc_counts(dst_hbm, typ_hbm, zcnt_hbm, cnt_out, dstb, typb, keyb, onesb,
               shared_cnt):
    c = lax.axis_index("c")
    s = lax.axis_index("s")
    wid = s * 2 + c
    for j in range(CH // 16):
        onesb[pl.ds(j * 16, 16)] = jnp.full((16,), 1.0, jnp.float32)
    pltpu.sync_copy(zcnt_hbm, shared_cnt.at[pl.ds(s * (NRPAD // 16), NRPAD // 16)])
    plsc.subcore_barrier()

    def chunk(i, _):
        eb = wid * EPT32 + i * CH
        pltpu.sync_copy(dst_hbm.at[pl.ds(eb, CH)], dstb)
        pltpu.sync_copy(typ_hbm.at[pl.ds(eb, CH)], typb)
        for j in range(CH // 16):
            sl = pl.ds(j * 16, 16)
            keyb[sl] = dstb[sl] * R + typb[sl]
        pltpu.sync_copy(onesb, shared_cnt.at[keyb], add=True)
        return 0

    lax.fori_loop(0, EPT32 // CH, chunk, 0)
    plsc.subcore_barrier()
    sl = pl.ds(s * (NRPAD // 16), NRPAD // 16)
    pltpu.sync_copy(shared_cnt.at[sl], cnt_out.at[c, sl])


@functools.partial(
    pl.kernel,
    mesh=_MESH,
    out_type=_f32((NPAD, HPAD)),
    scratch_types=[
        pltpu.VMEM((CH,), jnp.int32),
        pltpu.VMEM((CH,), jnp.int32),
        pltpu.VMEM((CH,), jnp.int32),
        pltpu.VMEM((CH,), jnp.int32),
        pltpu.VMEM((CH,), jnp.int32),
        pltpu.VMEM((CH,), jnp.float32),
        pltpu.VMEM((CH, HH), jnp.float32),
        pltpu.SemaphoreType.DMA,
        pltpu.VMEM_SHARED((NAGG, HH), jnp.float32),
    ],
)
def _sc_aggregate(xtlo_hbm, xthi_hbm, norm_hbm, src_hbm, dst_hbm, typ_hbm,
                  zrow_hbm, agg_out, srcb, dstb, typb, gixb, keyb, nrmb,
                  rowsb, sem, shared_agg):
    c = lax.axis_index("c")
    s = lax.axis_index("s")

    @pl.when(s < 8)
    def _zero():
        pltpu.sync_copy(zrow_hbm, shared_agg.at[pl.ds(s * ROWS_PS, ROWS_PS)])

    plsc.subcore_barrier()

    def chunk(i, _):
        eb = s * EPT16 + i * CH
        pltpu.sync_copy(src_hbm.at[pl.ds(eb, CH)], srcb)
        pltpu.sync_copy(dst_hbm.at[pl.ds(eb, CH)], dstb)
        pltpu.sync_copy(typ_hbm.at[pl.ds(eb, CH)], typb)
        for j in range(CH // 16):
            sl = pl.ds(j * 16, 16)
            t = typb[sl]
            gixb[sl] = t * NPAD + srcb[sl]
            keyb[sl] = dstb[sl] * R + t

        @pl.when(c == 0)
        def _gather_lo():
            pltpu.async_copy(xtlo_hbm.at[gixb], rowsb, sem).wait()

        @pl.when(c == 1)
        def _gather_hi():
            pltpu.async_copy(xthi_hbm.at[gixb], rowsb, sem).wait()

        pltpu.async_copy(norm_hbm.at[keyb], nrmb, sem).wait()

        def scale_group(g, _):
            nv = nrmb[pl.ds(g * 16, 16)]
            for k in range(16):
                r = g * 16 + k
                nsplat = jnp.full((16,), nv[k], jnp.float32)
                for hs in range(HH // 16):
                    hsl = pl.ds(hs * 16, 16)
                    rowsb[r, hsl] = rowsb[r, hsl] * nsplat
            return 0

        lax.fori_loop(0, CH // 16, scale_group, 0)
        pltpu.sync_copy(rowsb, shared_agg.at[dstb], add=True)
        return 0

    lax.fori_loop(0, EPT16 // CH, chunk, 0)
    plsc.subcore_barrier()

    @pl.when(s < 8)
    def _copy_out():
        rsl = pl.ds(s * ROWS_PS, ROWS_PS)
        pltpu.sync_copy(shared_agg.at[rsl],
                        agg_out.at[rsl, pl.ds(c * HH, HH)])


@functools.partial(
    pl.kernel,
    mesh=_MESH,
    out_type=_f32((E * 16,)),
    scratch_types=[
        pltpu.VMEM((CH,), jnp.int32),
        pltpu.VMEM((CH,), jnp.int32),
        pltpu.VMEM((CH,), jnp.int32),
        pltpu.VMEM((CH, HPAD), jnp.float32),
        pltpu.VMEM((CH, HPAD), jnp.float32),
        pltpu.VMEM((CH, HPAD), jnp.float32),
        pltpu.VMEM((CH * 16,), jnp.float32),
        pltpu.SemaphoreType.DMA,
    ],
)
def _sc_decode(z_hbm, rel_hbm, src_hbm, dst_hbm, typ_hbm, part_out,
               srcb, dstb, typb, zsb, zdb, rlb, outb, sem):
    c = lax.axis_index("c")
    s = lax.axis_index("s")
    wid = s * 2 + c

    def chunk(i, _):
        eb = wid * EPT32 + i * CH
        pltpu.sync_copy(src_hbm.at[pl.ds(eb, CH)], srcb)
        pltpu.sync_copy(dst_hbm.at[pl.ds(eb, CH)], dstb)
        pltpu.sync_copy(typ_hbm.at[pl.ds(eb, CH)], typb)
        pltpu.async_copy(z_hbm.at[srcb], zsb, sem).wait()
        pltpu.async_copy(z_hbm.at[dstb], zdb, sem).wait()
        pltpu.async_copy(rel_hbm.at[typb], rlb, sem).wait()

        def edge(r, _):
            acc = (zsb[r, pl.ds(0, 16)] * zdb[r, pl.ds(0, 16)]
                   * rlb[r, pl.ds(0, 16)])
            for hs in range(1, NSL):
                hsl = pl.ds(hs * 16, 16)
                acc = acc + zsb[r, hsl] * zdb[r, hsl] * rlb[r, hsl]
            outb[pl.ds(r * 16, 16)] = acc
            return 0

        lax.fori_loop(0, CH, edge, 0)
        pltpu.sync_copy(outb, part_out.at[pl.ds(eb * 16, CH * 16)])
        return 0

    lax.fori_loop(0, EPT32 // CH, chunk, 0)


def _tc_norm_body(c_ref, o_ref):
    c = c_ref[0] + c_ref[1]
    o_ref[...] = 1.0 / jnp.maximum(c, 1.0)


def _tc_finish_body(p_ref, o_ref):
    rows = p_ref.shape[0]
    o_ref[...] = jnp.sum(p_ref[...].reshape(rows, 8, 16), axis=2)


def _tc_finish(part):
    rows = E * 16 // 128
    blk = 4000
    out = pl.pallas_call(
        _tc_finish_body,
        grid=(rows // blk,),
        in_specs=[pl.BlockSpec((blk, 128), lambda i: (i, 0))],
        out_specs=pl.BlockSpec((blk, 8), lambda i: (i, 0)),
        out_shape=_f32((rows, 8)),
    )(part.reshape(rows, 128))
    return out.reshape(E)


def _tc_transform_body(x_ref, w_ref, lo_ref, hi_ref):
    x = x_ref[...]
    for r in range(R):
        res = jnp.dot(x, w_ref[r], preferred_element_type=jnp.float32)
        lo_ref[r] = res[:, :HH]
        hi_ref[r] = res[:, HH:]


def _tc_root_body(agg_ref, x_ref, w_ref, b_ref, o_ref, *, relu):
    y = (agg_ref[...]
         + jnp.dot(x_ref[...], w_ref[...], preferred_element_type=jnp.float32)
         + b_ref[...])
    o_ref[...] = jnp.maximum(y, 0.0) if relu else y


def _tc_norm(cnt2):
    out = pl.pallas_call(
        _tc_norm_body,
        grid=(1,),
        in_specs=[pl.BlockSpec((2, NRPAD // 128, 128), lambda i: (0, 0, 0))],
        out_specs=pl.BlockSpec((NRPAD // 128, 128), lambda i: (0, 0)),
        out_shape=_f32((NRPAD // 128, 128)),
    )(cnt2.reshape(2, NRPAD // 128, 128))
    return out.reshape(NRPAD)


def _tc_transform(xp, wd):
    lo, hi = pl.pallas_call(
        _tc_transform_body,
        grid=(NPAD // 256,),
        in_specs=[
            pl.BlockSpec((256, HPAD), lambda i: (i, 0)),
            pl.BlockSpec((R, HPAD, HPAD), lambda i: (0, 0, 0)),
        ],
        out_specs=[
            pl.BlockSpec((R, 256, HH), lambda i: (0, i, 0)),
            pl.BlockSpec((R, 256, HH), lambda i: (0, i, 0)),
        ],
        out_shape=[_f32((R, NPAD, HH)), _f32((R, NPAD, HH))],
    )(xp, wd)
    return lo.reshape(R * NPAD, HH), hi.reshape(R * NPAD, HH)


def _tc_root(agg, xp, rootp, bp, relu):
    return pl.pallas_call(
        functools.partial(_tc_root_body, relu=relu),
        grid=(NPAD // 256,),
        in_specs=[
            pl.BlockSpec((256, HPAD), lambda i: (i, 0)),
            pl.BlockSpec((256, HPAD), lambda i: (i, 0)),
            pl.BlockSpec((HPAD, HPAD), lambda i: (0, 0)),
            pl.BlockSpec((1, HPAD), lambda i: (0, 0)),
        ],
        out_specs=pl.BlockSpec((256, HPAD), lambda i: (i, 0)),
        out_shape=_f32((NPAD, HPAD)),
    )(agg, xp, rootp, bp)


def _dense_blocks(Wb):
    wd = jnp.zeros((R, HPAD, HPAD), jnp.float32)
    for bi in range(NB):
        wd = wd.at[:, bi * BS:(bi + 1) * BS, bi * BS:(bi + 1) * BS].set(Wb[:, bi])
    return wd


def _pad_mat(m):
    out = jnp.zeros((HPAD, HPAD), jnp.float32)
    return out.at[:H, :H].set(m)


def kernel(node_emb, W1, root1, b1, W2, root2, b2, rel_emb, edge_index, edge_type):
    src = edge_index[0]
    dst = edge_index[1]
    typ = edge_type

    xp = jnp.zeros((NPAD, HPAD), jnp.float32).at[:N, :H].set(node_emb)
    wd1 = _dense_blocks(W1)
    wd2 = _dense_blocks(W2)
    root1p = _pad_mat(root1)
    root2p = _pad_mat(root2)
    b1p = jnp.zeros((1, HPAD), jnp.float32).at[0, :H].set(b1)
    b2p = jnp.zeros((1, HPAD), jnp.float32).at[0, :H].set(b2)
    relp = jnp.zeros((R, HPAD), jnp.float32).at[:, :H].set(rel_emb)
    zcnt = jnp.zeros((NRPAD // 16,), jnp.float32)
    zrow = jnp.zeros((ROWS_PS, HH), jnp.float32)

    cnt2 = _sc_counts(dst, typ, zcnt)
    norm = _tc_norm(cnt2)

    xt1lo, xt1hi = _tc_transform(xp, wd1)
    agg1 = _sc_aggregate(xt1lo, xt1hi, norm, src, dst, typ, zrow)
    h = _tc_root(agg1, xp, root1p, b1p, relu=True)

    xt2lo, xt2hi = _tc_transform(h, wd2)
    agg2 = _sc_aggregate(xt2lo, xt2hi, norm, src, dst, typ, zrow)
    z = _tc_root(agg2, h, root2p, b2p, relu=False)

    part = _sc_decode(z, relp, src, dst, typ)
    return _tc_finish(part)


# speedup vs baseline: 4.6856x; 1.6508x over previous
import functools

import jax
import jax.numpy as jnp
from jax import lax
from jax.experimental import pallas as pl
from jax.experimental.pallas import tpu as pltpu
from jax.experimental.pallas import tpu_sc as plsc

N = 10000
R = 30
H = 200
NB = 5
BS = 40
E = 320000

HPAD = 256
HH = 128
NPAD = 10240
NR = N * R
NRPAD = 307200
NTILES = 32
CH = 80
EPT32 = E // 32
EPT16 = E // 16
NAGG = 10048
ROWS_PS = 1256
NSL = 13
RPAD = 32

_MESH = plsc.VectorSubcoreMesh(core_axis_name="c", subcore_axis_name="s")


def _f32(shape):
    return jax.ShapeDtypeStruct(shape, jnp.float32)


@functools.partial(
    pl.kernel,
    mesh=_MESH,
    out_type=_f32((2, NRPAD)),
    scratch_types=[
        pltpu.VMEM((CH,), jnp.int32),
        pltpu.VMEM((CH,), jnp.int32),
        pltpu.VMEM((CH,), jnp.int32),
        pltpu.VMEM((CH,), jnp.float32),
        pltpu.VMEM_SHARED((NRPAD,), jnp.float32),
    ],
)
def _sc_counts(dst_hbm, typ_hbm, zcnt_hbm, cnt_out, dstb, typb, keyb, onesb,
               shared_cnt):
    c = lax.axis_index("c")
    s = lax.axis_index("s")
    wid = s * 2 + c
    for j in range(CH // 16):
        onesb[pl.ds(j * 16, 16)] = jnp.full((16,), 1.0, jnp.float32)
    pltpu.sync_copy(zcnt_hbm, shared_cnt.at[pl.ds(s * (NRPAD // 16), NRPAD // 16)])
    plsc.subcore_barrier()

    def chunk(i, _):
        eb = wid * EPT32 + i * CH
        pltpu.sync_copy(dst_hbm.at[pl.ds(eb, CH)], dstb)
        pltpu.sync_copy(typ_hbm.at[pl.ds(eb, CH)], typb)
        for j in range(CH // 16):
            sl = pl.ds(j * 16, 16)
            keyb[sl] = dstb[sl] * R + typb[sl]
        pltpu.sync_copy(onesb, shared_cnt.at[keyb], add=True)
        return 0

    lax.fori_loop(0, EPT32 // CH, chunk, 0)
    plsc.subcore_barrier()
    sl = pl.ds(s * (NRPAD // 16), NRPAD // 16)
    pltpu.sync_copy(shared_cnt.at[sl], cnt_out.at[c, sl])


@functools.partial(
    pl.kernel,
    mesh=_MESH,
    out_type=_f32((NPAD, HPAD)),
    scratch_types=[
        pltpu.VMEM((CH,), jnp.int32),
        pltpu.VMEM((CH,), jnp.int32),
        pltpu.VMEM((CH,), jnp.int32),
        pltpu.VMEM((CH,), jnp.int32),
        pltpu.VMEM((CH,), jnp.int32),
        pltpu.VMEM((CH,), jnp.int32),
        pltpu.VMEM((CH,), jnp.int32),
        pltpu.VMEM((CH,), jnp.int32),
        pltpu.VMEM((CH,), jnp.float32),
        pltpu.VMEM((CH,), jnp.float32),
        pltpu.VMEM((CH, HH), jnp.float32),
        pltpu.VMEM((CH, HH), jnp.float32),
        pltpu.SemaphoreType.DMA,
        pltpu.SemaphoreType.DMA,
        pltpu.VMEM_SHARED((NAGG, HH), jnp.float32),
    ],
)
def _sc_aggregate(xtlo_hbm, xthi_hbm, norm_hbm, src_hbm, dst_hbm, typ_hbm,
                  zrow_hbm, agg_out, srcb, typb, dstA, dstB, gixA, gixB,
                  keyA, keyB, nrmA, nrmB, rowsA, rowsB, semA, semB,
                  shared_agg):
    c = lax.axis_index("c")
    s = lax.axis_index("s")

    @pl.when(s < 8)
    def _zero():
        pltpu.sync_copy(zrow_hbm, shared_agg.at[pl.ds(s * ROWS_PS, ROWS_PS)])

    plsc.subcore_barrier()

    def stage(i, dstb, gixb, keyb, nrmb, rowsb, sem):
        eb = s * EPT16 + i * CH
        pltpu.sync_copy(src_hbm.at[pl.ds(eb, CH)], srcb)
        pltpu.sync_copy(dst_hbm.at[pl.ds(eb, CH)], dstb)
        pltpu.sync_copy(typ_hbm.at[pl.ds(eb, CH)], typb)
        for j in range(CH // 16):
            sl = pl.ds(j * 16, 16)
            t = typb[sl]
            gixb[sl] = t * NPAD + srcb[sl]
            keyb[sl] = dstb[sl] * R + t

        @pl.when(c == 0)
        def _gather_lo():
            pltpu.async_copy(xtlo_hbm.at[gixb], rowsb, sem)

        @pl.when(c == 1)
        def _gather_hi():
            pltpu.async_copy(xthi_hbm.at[gixb], rowsb, sem)

        pltpu.async_copy(norm_hbm.at[keyb], nrmb, sem)

    def finish(dstb, gixb, keyb, nrmb, rowsb, sem):
        pltpu.make_async_copy(xtlo_hbm.at[gixb], rowsb, sem).wait()
        pltpu.make_async_copy(norm_hbm.at[keyb], nrmb, sem).wait()

        def scale_group(g, _):
            nv = nrmb[pl.ds(g * 16, 16)]
            for k in range(16):
                r = g * 16 + k
                nsplat = jnp.full((16,), nv[k], jnp.float32)
                for hs in range(HH // 16):
                    hsl = pl.ds(hs * 16, 16)
                    rowsb[r, hsl] = rowsb[r, hsl] * nsplat
            return 0

        lax.fori_loop(0, CH // 16, scale_group, 0)
        pltpu.sync_copy(rowsb, shared_agg.at[dstb], add=True)

    nhalf = EPT16 // CH // 2
    stage(0, dstA, gixA, keyA, nrmA, rowsA, semA)

    def body(i2, _):
        iA = i2 * 2
        stage(iA + 1, dstB, gixB, keyB, nrmB, rowsB, semB)
        finish(dstA, gixA, keyA, nrmA, rowsA, semA)

        @pl.when(i2 + 1 < nhalf)
        def _prefetch():
            stage(iA + 2, dstA, gixA, keyA, nrmA, rowsA, semA)

        finish(dstB, gixB, keyB, nrmB, rowsB, semB)
        return 0

    lax.fori_loop(0, nhalf, body, 0)
    plsc.subcore_barrier()

    @pl.when(s < 8)
    def _copy_out():
        rsl = pl.ds(s * ROWS_PS, ROWS_PS)
        pltpu.sync_copy(shared_agg.at[rsl],
                        agg_out.at[rsl, pl.ds(c * HH, HH)])


@functools.partial(
    pl.kernel,
    mesh=_MESH,
    out_type=_f32((E * 16,)),
    scratch_types=[
        pltpu.VMEM((CH,), jnp.int32),
        pltpu.VMEM((CH,), jnp.int32),
        pltpu.VMEM((CH,), jnp.int32),
        pltpu.VMEM((CH,), jnp.int32),
        pltpu.VMEM((CH,), jnp.int32),
        pltpu.VMEM((CH,), jnp.int32),
        pltpu.VMEM((CH, HPAD), jnp.float32),
        pltpu.VMEM((CH, HPAD), jnp.float32),
        pltpu.VMEM((CH, HPAD), jnp.float32),
        pltpu.VMEM((CH, HPAD), jnp.float32),
        pltpu.VMEM((RPAD, HPAD), jnp.float32),
        pltpu.VMEM((CH * 16,), jnp.float32),
        pltpu.SemaphoreType.DMA,
        pltpu.SemaphoreType.DMA,
    ],
)
def _sc_decode(z_hbm, rel_hbm, src_hbm, dst_hbm, typ_hbm, part_out,
               srcA, srcB, dstA, dstB, typA, typB, zsA, zsB, zdA, zdB,
               rlv, outb, semA, semB):
    c = lax.axis_index("c")
    s = lax.axis_index("s")
    wid = s * 2 + c
    pltpu.sync_copy(rel_hbm, rlv)

    def stage(i, srcb, dstb, typb, zsb, zdb, sem):
        eb = wid * EPT32 + i * CH
        pltpu.sync_copy(src_hbm.at[pl.ds(eb, CH)], srcb)
        pltpu.sync_copy(dst_hbm.at[pl.ds(eb, CH)], dstb)
        pltpu.sync_copy(typ_hbm.at[pl.ds(eb, CH)], typb)
        pltpu.async_copy(z_hbm.at[srcb], zsb, sem)
        pltpu.async_copy(z_hbm.at[dstb], zdb, sem)

    def finish(i, srcb, dstb, typb, zsb, zdb, sem):
        eb = wid * EPT32 + i * CH
        pltpu.make_async_copy(z_hbm.at[srcb], zsb, sem).wait()
        pltpu.make_async_copy(z_hbm.at[dstb], zdb, sem).wait()

        def group(g, _):
            tv = typb[pl.ds(g * 16, 16)]
            for k in range(16):
                r = g * 16 + k
                t = tv[k]
                acc = (zsb[r, pl.ds(0, 16)] * zdb[r, pl.ds(0, 16)]
                       * rlv[t, pl.ds(0, 16)])
                for hs in range(1, NSL):
                    hsl = pl.ds(hs * 16, 16)
                    acc = acc + zsb[r, hsl] * zdb[r, hsl] * rlv[t, hsl]
                outb[pl.ds(r * 16, 16)] = acc
            return 0

        lax.fori_loop(0, CH // 16, group, 0)
        pltpu.sync_copy(outb, part_out.at[pl.ds(eb * 16, CH * 16)])

    nhalf = EPT32 // CH // 2
    stage(0, srcA, dstA, typA, zsA, zdA, semA)

    def body(i2, _):
        iA = i2 * 2
        stage(iA + 1, srcB, dstB, typB, zsB, zdB, semB)
        finish(iA, srcA, dstA, typA, zsA, zdA, semA)

        @pl.when(i2 + 1 < nhalf)
        def _prefetch():
            stage(iA + 2, srcA, dstA, typA, zsA, zdA, semA)

        finish(iA + 1, srcB, dstB, typB, zsB, zdB, semB)
        return 0

    lax.fori_loop(0, nhalf, body, 0)
    last = EPT32 // CH - 1
    stage(last, srcA, dstA, typA, zsA, zdA, semA)
    finish(last, srcA, dstA, typA, zsA, zdA, semA)


def _tc_norm_body(c_ref, o_ref):
    c = c_ref[0] + c_ref[1]
    o_ref[...] = 1.0 / jnp.maximum(c, 1.0)


def _tc_finish_body(p_ref, o_ref):
    rows = p_ref.shape[0]
    o_ref[...] = jnp.sum(p_ref[...].reshape(rows, 8, 16), axis=2)


def _tc_finish(part):
    rows = E * 16 // 128
    blk = 4000
    out = pl.pallas_call(
        _tc_finish_body,
        grid=(rows // blk,),
        in_specs=[pl.BlockSpec((blk, 128), lambda i: (i, 0))],
        out_specs=pl.BlockSpec((blk, 8), lambda i: (i, 0)),
        out_shape=_f32((rows, 8)),
    )(part.reshape(rows, 128))
    return out.reshape(E)


def _tc_transform_body(x_ref, w_ref, lo_ref, hi_ref):
    x = x_ref[...]
    for r in range(R):
        res = jnp.dot(x, w_ref[r], preferred_element_type=jnp.float32)
        lo_ref[r] = res[:, :HH]
        hi_ref[r] = res[:, HH:]


def _tc_root_body(agg_ref, x_ref, w_ref, b_ref, o_ref, *, relu):
    y = (agg_ref[...]
         + jnp.dot(x_ref[...], w_ref[...], preferred_element_type=jnp.float32)
         + b_ref[...])
    o_ref[...] = jnp.maximum(y, 0.0) if relu else y


def _tc_norm(cnt2):
    out = pl.pallas_call(
        _tc_norm_body,
        grid=(1,),
        in_specs=[pl.BlockSpec((2, NRPAD // 128, 128), lambda i: (0, 0, 0))],
        out_specs=pl.BlockSpec((NRPAD // 128, 128), lambda i: (0, 0)),
        out_shape=_f32((NRPAD // 128, 128)),
    )(cnt2.reshape(2, NRPAD // 128, 128))
    return out.reshape(NRPAD)


def _tc_transform(xp, wd):
    lo, hi = pl.pallas_call(
        _tc_transform_body,
        grid=(NPAD // 256,),
        in_specs=[
            pl.BlockSpec((256, HPAD), lambda i: (i, 0)),
            pl.BlockSpec((R, HPAD, HPAD), lambda i: (0, 0, 0)),
        ],
        out_specs=[
            pl.BlockSpec((R, 256, HH), lambda i: (0, i, 0)),
            pl.BlockSpec((R, 256, HH), lambda i: (0, i, 0)),
        ],
        out_shape=[_f32((R, NPAD, HH)), _f32((R, NPAD, HH))],
    )(xp, wd)
    return lo.reshape(R * NPAD, HH), hi.reshape(R * NPAD, HH)


def _tc_root(agg, xp, rootp, bp, relu):
    return pl.pallas_call(
        functools.partial(_tc_root_body, relu=relu),
        grid=(NPAD // 256,),
        in_specs=[
            pl.BlockSpec((256, HPAD), lambda i: (i, 0)),
            pl.BlockSpec((256, HPAD), lambda i: (i, 0)),
            pl.BlockSpec((HPAD, HPAD), lambda i: (0, 0)),
            pl.BlockSpec((1, HPAD), lambda i: (0, 0)),
        ],
        out_specs=pl.BlockSpec((256, HPAD), lambda i: (i, 0)),
        out_shape=_f32((NPAD, HPAD)),
    )(agg, xp, rootp, bp)


def _dense_blocks(Wb):
    wd = jnp.zeros((R, HPAD, HPAD), jnp.float32)
    for bi in range(NB):
        wd = wd.at[:, bi * BS:(bi + 1) * BS, bi * BS:(bi + 1) * BS].set(Wb[:, bi])
    return wd


def _pad_mat(m):
    out = jnp.zeros((HPAD, HPAD), jnp.float32)
    return out.at[:H, :H].set(m)


def kernel(node_emb, W1, root1, b1, W2, root2, b2, rel_emb, edge_index, edge_type):
    src = edge_index[0]
    dst = edge_index[1]
    typ = edge_type

    xp = jnp.zeros((NPAD, HPAD), jnp.float32).at[:N, :H].set(node_emb)
    wd1 = _dense_blocks(W1)
    wd2 = _dense_blocks(W2)
    root1p = _pad_mat(root1)
    root2p = _pad_mat(root2)
    b1p = jnp.zeros((1, HPAD), jnp.float32).at[0, :H].set(b1)
    b2p = jnp.zeros((1, HPAD), jnp.float32).at[0, :H].set(b2)
    relp = jnp.zeros((RPAD, HPAD), jnp.float32).at[:R, :H].set(rel_emb)
    zcnt = jnp.zeros((NRPAD // 16,), jnp.float32)
    zrow = jnp.zeros((ROWS_PS, HH), jnp.float32)

    cnt2 = _sc_counts(dst, typ, zcnt)
    norm = _tc_norm(cnt2)

    xt1lo, xt1hi = _tc_transform(xp, wd1)
    agg1 = _sc_aggregate(xt1lo, xt1hi, norm, src, dst, typ, zrow)
    h = _tc_root(agg1, xp, root1p, b1p, relu=True)

    xt2lo, xt2hi = _tc_transform(h, wd2)
    agg2 = _sc_aggregate(xt2lo, xt2hi, norm, src, dst, typ, zrow)
    z = _tc_root(agg2, h, root2p, b2p, relu=False)

    part = _sc_decode(z, relp, src, dst, typ)
    return _tc_finish(part)
